# Initial kernel scaffold; baseline (speedup 1.0000x reference)
import jax
import jax.numpy as jnp
from jax.experimental import pallas as pl


def kernel(reactant_ids, template_ids, r_table, t_table, W_m, b_m, W_t, b_t):
    raise NotImplementedError("write your pallas kernel here")


# speedup vs baseline: 12.9563x; 12.9563x over previous
import functools

import jax
import jax.numpy as jnp
from jax import lax
from jax.experimental import pallas as pl
from jax.experimental.pallas import tpu as pltpu
from jax.experimental.pallas import tpu_sc as plsc

B = 16384
R = 7
H = 128

_info = plsc.get_sparse_core_info()
_NC, _NS, _L = _info.num_cores, _info.num_subcores, _info.num_lanes
NW = _NC * _NS
TREES_PER_W = B // NW
CHUNK = 16
NCHUNK = TREES_PER_W // CHUNK


def _sc_body(rids_hbm, tids_hbm, r_table, t_table, msum_hbm, tfeat_hbm,
             ridx_v, rrows_v, tidx_v, trows_v, msum_v, sem):
    wid = lax.axis_index("s") * _NC + lax.axis_index("c")

    def chunk_body(i, carry):
        base = wid * TREES_PER_W + i * CHUNK
        pltpu.sync_copy(rids_hbm.at[pl.ds(base * R, CHUNK * R)], ridx_v)
        pltpu.sync_copy(tids_hbm.at[pl.ds(base, CHUNK)], tidx_v)
        rcp = pltpu.async_copy(r_table.at[ridx_v], rrows_v, sem)
        tcp = pltpu.async_copy(t_table.at[tidx_v], trows_v, sem)
        rcp.wait()
        tcp.wait()

        def tree_body(t, c2):
            row = t * R
            for c in range(H // _L):
                acc = rrows_v[row, pl.ds(c * _L, _L)]
                for j in range(1, R):
                    acc = acc + rrows_v[row + j, pl.ds(c * _L, _L)]
                msum_v[t, pl.ds(c * _L, _L)] = acc
            return c2

        lax.fori_loop(0, CHUNK, tree_body, 0, unroll=2)
        pltpu.sync_copy(msum_v, msum_hbm.at[pl.ds(base, CHUNK)])
        pltpu.sync_copy(trows_v, tfeat_hbm.at[pl.ds(base, CHUNK)])
        return carry

    lax.fori_loop(0, NCHUNK, chunk_body, 0)


_sc_pool = functools.partial(
    pl.kernel,
    out_type=(
        jax.ShapeDtypeStruct((B, H), jnp.float32),
        jax.ShapeDtypeStruct((B, H), jnp.float32),
    ),
    mesh=plsc.VectorSubcoreMesh(core_axis_name="c", subcore_axis_name="s"),
    scratch_types=[
        pltpu.VMEM((CHUNK * R,), jnp.int32),
        pltpu.VMEM((CHUNK * R, H), jnp.float32),
        pltpu.VMEM((CHUNK,), jnp.int32),
        pltpu.VMEM((CHUNK, H), jnp.float32),
        pltpu.VMEM((CHUNK, H), jnp.float32),
        pltpu.SemaphoreType.DMA,
    ],
)(_sc_body)


def _tc_body(ms_ref, tf_ref, wm_ref, wt_ref, bm_ref, bt_ref, o_ref):
    acc = jnp.dot(ms_ref[...], wm_ref[...], preferred_element_type=jnp.float32)
    acc = acc + jnp.dot(tf_ref[...], wt_ref[...],
                        preferred_element_type=jnp.float32)
    o_ref[...] = jnp.maximum(acc + 7.0 * bm_ref[...] + bt_ref[...], 0.0)


def _tc_combine(msum, tfeat, W_m, W_t, b_m2, b_t2):
    BM = 2048
    return pl.pallas_call(
        _tc_body,
        grid=(B // BM,),
        in_specs=[
            pl.BlockSpec((BM, H), lambda i: (i, 0)),
            pl.BlockSpec((BM, H), lambda i: (i, 0)),
            pl.BlockSpec((H, H), lambda i: (0, 0)),
            pl.BlockSpec((H, H), lambda i: (0, 0)),
            pl.BlockSpec((1, H), lambda i: (0, 0)),
            pl.BlockSpec((1, H), lambda i: (0, 0)),
        ],
        out_specs=pl.BlockSpec((BM, H), lambda i: (i, 0)),
        out_shape=jax.ShapeDtypeStruct((B, H), jnp.float32),
    )(msum, tfeat, W_m, W_t, b_m2, b_t2)


def kernel(reactant_ids, template_ids, r_table, t_table, W_m, b_m, W_t, b_t):
    rids = reactant_ids.astype(jnp.int32).reshape(-1)
    tids = template_ids.astype(jnp.int32)
    msum, tfeat = _sc_pool(rids, tids, r_table, t_table)
    return _tc_combine(msum, tfeat, W_m, W_t,
                       b_m.reshape(1, H), b_t.reshape(1, H))


# speedup vs baseline: 13.1513x; 1.0150x over previous
import functools

import jax
import jax.numpy as jnp
from jax import lax
from jax.experimental import pallas as pl
from jax.experimental.pallas import tpu as pltpu
from jax.experimental.pallas import tpu_sc as plsc

B = 16384
R = 7
H = 128

_info = plsc.get_sparse_core_info()
_NC, _NS, _L = _info.num_cores, _info.num_subcores, _info.num_lanes
NW = _NC * _NS
TREES_PER_W = B // NW
CHUNK = 8
NCHUNK = TREES_PER_W // CHUNK
NBUF = 2


def _sc_body(rids_hbm, tids_hbm, r_table, t_table, msum_hbm, tfeat_hbm,
             ridx_v, rrows_v, tidx_v, trows_v, msum_v,
             rsem0, rsem1, tsem0, tsem1):
    wid = lax.axis_index("s") * _NC + lax.axis_index("c")
    wbase = wid * TREES_PER_W
    rsems = (rsem0, rsem1)
    tsems = (tsem0, tsem1)

    def start(i, b):
        base = wbase + i * CHUNK
        pltpu.sync_copy(rids_hbm.at[pl.ds(base * R, CHUNK * R)], ridx_v.at[b])
        pltpu.sync_copy(tids_hbm.at[pl.ds(base, CHUNK)], tidx_v.at[b])
        pltpu.async_copy(r_table.at[ridx_v.at[b]], rrows_v.at[b], rsems[b])
        pltpu.async_copy(t_table.at[tidx_v.at[b]], trows_v.at[b], tsems[b])

    def wait(b):
        pltpu.make_async_copy(r_table.at[ridx_v.at[b]], rrows_v.at[b],
                              rsems[b]).wait()
        pltpu.make_async_copy(t_table.at[tidx_v.at[b]], trows_v.at[b],
                              tsems[b]).wait()

    def accumulate(b):
        for t in range(CHUNK):
            row = t * R
            for c in range(H // _L):
                sl = pl.ds(c * _L, _L)
                a01 = rrows_v[b, row + 0, sl] + rrows_v[b, row + 1, sl]
                a23 = rrows_v[b, row + 2, sl] + rrows_v[b, row + 3, sl]
                a45 = rrows_v[b, row + 4, sl] + rrows_v[b, row + 5, sl]
                msum_v[t, sl] = (a01 + a23) + (a45 + rrows_v[b, row + 6, sl])

    start(0, 0)

    def outer(g, carry):
        for b in range(NBUF):
            i = g * NBUF + b
            wait(b)

            @pl.when(i + 1 < NCHUNK)
            def _():
                start(i + 1, (b + 1) % NBUF)

            accumulate(b)
            base = wbase + i * CHUNK
            pltpu.sync_copy(msum_v, msum_hbm.at[pl.ds(base, CHUNK)])
            pltpu.sync_copy(trows_v.at[b], tfeat_hbm.at[pl.ds(base, CHUNK)])
        return carry

    lax.fori_loop(0, NCHUNK // NBUF, outer, 0)


_sc_pool = functools.partial(
    pl.kernel,
    out_type=(
        jax.ShapeDtypeStruct((B, H), jnp.float32),
        jax.ShapeDtypeStruct((B, H), jnp.float32),
    ),
    mesh=plsc.VectorSubcoreMesh(core_axis_name="c", subcore_axis_name="s"),
    scratch_types=[
        pltpu.VMEM((NBUF, CHUNK * R), jnp.int32),
        pltpu.VMEM((NBUF, CHUNK * R, H), jnp.float32),
        pltpu.VMEM((NBUF, CHUNK), jnp.int32),
        pltpu.VMEM((NBUF, CHUNK, H), jnp.float32),
        pltpu.VMEM((CHUNK, H), jnp.float32),
        pltpu.SemaphoreType.DMA,
        pltpu.SemaphoreType.DMA,
        pltpu.SemaphoreType.DMA,
        pltpu.SemaphoreType.DMA,
    ],
)(_sc_body)


def _tc_body(ms_ref, tf_ref, wm_ref, wt_ref, bm_ref, bt_ref, o_ref):
    acc = jnp.dot(ms_ref[...], wm_ref[...], preferred_element_type=jnp.float32)
    acc = acc + jnp.dot(tf_ref[...], wt_ref[...],
                        preferred_element_type=jnp.float32)
    o_ref[...] = jnp.maximum(acc + 7.0 * bm_ref[...] + bt_ref[...], 0.0)


def _tc_combine(msum, tfeat, W_m, W_t, b_m2, b_t2):
    BM = 2048
    return pl.pallas_call(
        _tc_body,
        grid=(B // BM,),
        in_specs=[
            pl.BlockSpec((BM, H), lambda i: (i, 0)),
            pl.BlockSpec((BM, H), lambda i: (i, 0)),
            pl.BlockSpec((H, H), lambda i: (0, 0)),
            pl.BlockSpec((H, H), lambda i: (0, 0)),
            pl.BlockSpec((1, H), lambda i: (0, 0)),
            pl.BlockSpec((1, H), lambda i: (0, 0)),
        ],
        out_specs=pl.BlockSpec((BM, H), lambda i: (i, 0)),
        out_shape=jax.ShapeDtypeStruct((B, H), jnp.float32),
    )(msum, tfeat, W_m, W_t, b_m2, b_t2)


def kernel(reactant_ids, template_ids, r_table, t_table, W_m, b_m, W_t, b_t):
    rids = reactant_ids.astype(jnp.int32).reshape(-1)
    tids = template_ids.astype(jnp.int32)
    msum, tfeat = _sc_pool(rids, tids, r_table, t_table)
    return _tc_combine(msum, tfeat, W_m, W_t,
                       b_m.reshape(1, H), b_t.reshape(1, H))


# speedup vs baseline: 15.6567x; 1.1905x over previous
import functools

import jax
import jax.numpy as jnp
from jax import lax
from jax.experimental import pallas as pl
from jax.experimental.pallas import tpu as pltpu
from jax.experimental.pallas import tpu_sc as plsc

B = 16384
R = 7
H = 128

_info = plsc.get_sparse_core_info()
_NC, _NS, _L = _info.num_cores, _info.num_subcores, _info.num_lanes
NW = _NC * _NS
TREES_PER_W = B // NW
CHUNK = 16
NCHUNK = TREES_PER_W // CHUNK
NBUF = 2


def _sc_body(rids_hbm, tids_hbm, r_table, t_table, msum_hbm, tfeat_hbm,
             ridx_all, tidx_all, rrows_v, trows_v, msum_v,
             rsem0, rsem1, tsem0, tsem1, wmsem0, wmsem1, wtsem0, wtsem1):
    wid = lax.axis_index("s") * _NC + lax.axis_index("c")
    wbase = wid * TREES_PER_W
    rsems = (rsem0, rsem1)
    tsems = (tsem0, tsem1)
    wmsems = (wmsem0, wmsem1)
    wtsems = (wtsem0, wtsem1)

    pltpu.sync_copy(rids_hbm.at[pl.ds(wid * NCHUNK, NCHUNK)], ridx_all)
    pltpu.sync_copy(tids_hbm.at[pl.ds(wid * NCHUNK, NCHUNK)], tidx_all)

    def start_gather(i, b):
        pltpu.async_copy(r_table.at[ridx_all.at[i]], rrows_v.at[b], rsems[b])
        pltpu.async_copy(t_table.at[tidx_all.at[i]], trows_v.at[b], tsems[b])

    def wait_gather(i, b):
        pltpu.make_async_copy(r_table.at[ridx_all.at[i]], rrows_v.at[b],
                              rsems[b]).wait()
        pltpu.make_async_copy(t_table.at[tidx_all.at[i]], trows_v.at[b],
                              tsems[b]).wait()

    def wait_writes(i, b):
        base = wbase + i * CHUNK
        pltpu.make_async_copy(msum_v.at[b], msum_hbm.at[pl.ds(base, CHUNK)],
                              wmsems[b]).wait()
        pltpu.make_async_copy(trows_v.at[b], tfeat_hbm.at[pl.ds(base, CHUNK)],
                              wtsems[b]).wait()

    def accumulate(b):
        for t in range(CHUNK):
            row = t * R
            for c in range(H // _L):
                sl = pl.ds(c * _L, _L)
                a01 = rrows_v[b, row + 0, sl] + rrows_v[b, row + 1, sl]
                a23 = rrows_v[b, row + 2, sl] + rrows_v[b, row + 3, sl]
                a45 = rrows_v[b, row + 4, sl] + rrows_v[b, row + 5, sl]
                msum_v[b, t, sl] = (a01 + a23) + (a45 + rrows_v[b, row + 6, sl])

    start_gather(0, 0)

    def outer(g, carry):
        for b in range(NBUF):
            i = g * NBUF + b
            b1 = (b + 1) % NBUF
            wait_gather(i, b)

            @pl.when(i + 1 < NCHUNK)
            def _():
                @pl.when(i >= 1)
                def _():
                    wait_writes(i - 1, b1)

                start_gather(i + 1, b1)

            accumulate(b)
            base = wbase + i * CHUNK
            pltpu.async_copy(msum_v.at[b], msum_hbm.at[pl.ds(base, CHUNK)],
                             wmsems[b])
            pltpu.async_copy(trows_v.at[b], tfeat_hbm.at[pl.ds(base, CHUNK)],
                             wtsems[b])
        return carry

    lax.fori_loop(0, NCHUNK // NBUF, outer, 0)
    wait_writes(NCHUNK - 2, (NCHUNK - 2) % NBUF)
    wait_writes(NCHUNK - 1, (NCHUNK - 1) % NBUF)


_sc_pool = functools.partial(
    pl.kernel,
    out_type=(
        jax.ShapeDtypeStruct((B, H), jnp.float32),
        jax.ShapeDtypeStruct((B, H), jnp.float32),
    ),
    mesh=plsc.VectorSubcoreMesh(core_axis_name="c", subcore_axis_name="s"),
    scratch_types=[
        pltpu.VMEM((NCHUNK, CHUNK * R), jnp.int32),
        pltpu.VMEM((NCHUNK, CHUNK), jnp.int32),
        pltpu.VMEM((NBUF, CHUNK * R, H), jnp.float32),
        pltpu.VMEM((NBUF, CHUNK, H), jnp.float32),
        pltpu.VMEM((NBUF, CHUNK, H), jnp.float32),
        pltpu.SemaphoreType.DMA,
        pltpu.SemaphoreType.DMA,
        pltpu.SemaphoreType.DMA,
        pltpu.SemaphoreType.DMA,
        pltpu.SemaphoreType.DMA,
        pltpu.SemaphoreType.DMA,
        pltpu.SemaphoreType.DMA,
        pltpu.SemaphoreType.DMA,
    ],
)(_sc_body)


def _tc_body(ms_ref, tf_ref, wm_ref, wt_ref, bm_ref, bt_ref, o_ref):
    acc = jnp.dot(ms_ref[...], wm_ref[...], preferred_element_type=jnp.float32)
    acc = acc + jnp.dot(tf_ref[...], wt_ref[...],
                        preferred_element_type=jnp.float32)
    o_ref[...] = jnp.maximum(acc + 7.0 * bm_ref[...] + bt_ref[...], 0.0)


def _tc_combine(msum, tfeat, W_m, W_t, b_m2, b_t2):
    BM = 2048
    return pl.pallas_call(
        _tc_body,
        grid=(B // BM,),
        in_specs=[
            pl.BlockSpec((BM, H), lambda i: (i, 0)),
            pl.BlockSpec((BM, H), lambda i: (i, 0)),
            pl.BlockSpec((H, H), lambda i: (0, 0)),
            pl.BlockSpec((H, H), lambda i: (0, 0)),
            pl.BlockSpec((1, H), lambda i: (0, 0)),
            pl.BlockSpec((1, H), lambda i: (0, 0)),
        ],
        out_specs=pl.BlockSpec((BM, H), lambda i: (i, 0)),
        out_shape=jax.ShapeDtypeStruct((B, H), jnp.float32),
    )(msum, tfeat, W_m, W_t, b_m2, b_t2)


def kernel(reactant_ids, template_ids, r_table, t_table, W_m, b_m, W_t, b_t):
    rids = reactant_ids.astype(jnp.int32).reshape(NW * NCHUNK, CHUNK * R)
    tids = template_ids.astype(jnp.int32).reshape(NW * NCHUNK, CHUNK)
    msum, tfeat = _sc_pool(rids, tids, r_table, t_table)
    return _tc_combine(msum, tfeat, W_m, W_t,
                       b_m.reshape(1, H), b_t.reshape(1, H))
